# baseline (speedup 1.0000x reference)
import functools
import jax
import jax.numpy as jnp
from jax import lax
from jax.experimental import pallas as pl
from jax.experimental.pallas import tpu as pltpu
from jax.experimental.pallas import tpu_sc as plsc

_ROWS = 4096
_COLS = 2048
_NW = 32
_B_CH = 2 * _ROWS // _NW
_CH = 8
_K = _B_CH // _CH
_NB = 7
_TILE = 512

_mesh = plsc.VectorSubcoreMesh(core_axis_name="c", subcore_axis_name="s")


@functools.partial(
    pl.kernel,
    mesh=_mesh,
    out_type=jax.ShapeDtypeStruct((3 * _ROWS, _COLS), jnp.float32),
    scratch_types=(
        [pltpu.VMEM((_CH, _COLS), jnp.float32)] * _NB
        + [pltpu.SemaphoreType.DMA] * (2 * _NB)
    ),
)
def _sc_compact(x_hbm, o_hbm, *scratch):
    bufs = scratch[:_NB]
    isems = scratch[_NB:2 * _NB]
    osems = scratch[2 * _NB:]
    wid = lax.axis_index("c") * 16 + lax.axis_index("s")
    b_dst = _ROWS + wid * _B_CH
    b_src = 2 * _ROWS + wid * _B_CH

    def in_copy(k):
        s = k % _NB
        return pltpu.async_copy(
            x_hbm.at[pl.ds(b_src + k * _CH, _CH)], bufs[s], isems[s]
        )

    def out_copy(k):
        s = k % _NB
        return pltpu.async_copy(
            bufs[s], o_hbm.at[pl.ds(b_dst + k * _CH, _CH)], osems[s]
        )

    h_in = {k: in_copy(k) for k in range(min(_NB, _K))}
    h_out = {}
    for k in range(_K):
        h_in[k].wait()
        h_out[k] = out_copy(k)
        if k + _NB < _K:
            h_out[k].wait()
            h_in[k + _NB] = in_copy(k + _NB)
    for k in range(max(0, _K - _NB), _K):
        h_out[k].wait()


def _tc_identity_body(full_ref, x_ref, o_ref):
    del full_ref
    o_ref[...] = x_ref[...]


def _tc_fill_identity(full, x2):
    return pl.pallas_call(
        _tc_identity_body,
        out_shape=jax.ShapeDtypeStruct((3 * _ROWS, _COLS), jnp.float32),
        grid=(_ROWS // _TILE,),
        in_specs=[
            pl.BlockSpec(memory_space=pl.ANY),
            pl.BlockSpec((_TILE, _COLS), lambda i: (i, 0)),
        ],
        out_specs=pl.BlockSpec((_TILE, _COLS), lambda i: (i, 0)),
        input_output_aliases={0: 0},
    )(full, x2)


def kernel(x):
    x2 = x.reshape(5 * _ROWS, _COLS)
    full = _sc_compact(x2)
    out = _tc_fill_identity(full, x2)
    return out.reshape(3, _ROWS, _COLS)


# speedup vs baseline: 1.0222x; 1.0222x over previous
import functools
import jax
import jax.numpy as jnp
from jax import lax
from jax.experimental import pallas as pl
from jax.experimental.pallas import tpu as pltpu
from jax.experimental.pallas import tpu_sc as plsc

_ROWS = 4096
_COLS = 2048
_NW = 32
_B_CH = 2 * _ROWS // _NW
_CH = 16
_K = _B_CH // _CH
_NB = 3
_TILE = 1024

_mesh = plsc.VectorSubcoreMesh(core_axis_name="c", subcore_axis_name="s")


@functools.partial(
    pl.kernel,
    mesh=_mesh,
    out_type=jax.ShapeDtypeStruct((3 * _ROWS, _COLS), jnp.float32),
    scratch_types=(
        [pltpu.VMEM((_CH, _COLS), jnp.float32)] * _NB
        + [pltpu.SemaphoreType.DMA] * (2 * _NB)
    ),
)
def _sc_compact(x_hbm, o_hbm, *scratch):
    bufs = scratch[:_NB]
    isems = scratch[_NB:2 * _NB]
    osems = scratch[2 * _NB:]
    wid = lax.axis_index("c") * 16 + lax.axis_index("s")
    b_dst = _ROWS + wid * _B_CH
    b_src = 2 * _ROWS + wid * _B_CH

    def in_copy(k):
        s = k % _NB
        return pltpu.async_copy(
            x_hbm.at[pl.ds(b_src + k * _CH, _CH)], bufs[s], isems[s]
        )

    def out_copy(k):
        s = k % _NB
        return pltpu.async_copy(
            bufs[s], o_hbm.at[pl.ds(b_dst + k * _CH, _CH)], osems[s]
        )

    h_in = {k: in_copy(k) for k in range(min(_NB, _K))}
    h_out = {}
    for k in range(_K):
        h_in[k].wait()
        h_out[k] = out_copy(k)
        if k + _NB < _K:
            h_out[k].wait()
            h_in[k + _NB] = in_copy(k + _NB)
    for k in range(max(0, _K - _NB), _K):
        h_out[k].wait()


def _tc_identity_body(full_ref, x_ref, o_ref):
    del full_ref
    o_ref[...] = x_ref[...]


def _tc_fill_identity(full, x2):
    return pl.pallas_call(
        _tc_identity_body,
        out_shape=jax.ShapeDtypeStruct((3 * _ROWS, _COLS), jnp.float32),
        grid=(_ROWS // _TILE,),
        in_specs=[
            pl.BlockSpec(memory_space=pl.ANY),
            pl.BlockSpec((_TILE, _COLS), lambda i: (i, 0)),
        ],
        out_specs=pl.BlockSpec((_TILE, _COLS), lambda i: (i, 0)),
        input_output_aliases={0: 0},
    )(full, x2)


def kernel(x):
    x2 = x.reshape(5 * _ROWS, _COLS)
    full = _sc_compact(x2)
    out = _tc_fill_identity(full, x2)
    return out.reshape(3, _ROWS, _COLS)
